# baseline (speedup 1.0000x reference)
import jax
import jax.numpy as jnp
from jax.experimental import pallas as pl
from jax.experimental.pallas import tpu as pltpu

_NBUF = 4


def _pick_block(dim, preferred):
    for b in preferred:
        if dim % b == 0:
            return b
    return dim


def _mm_stream_kernel(adj_hbm, x_ref, o_ref, buf, sems):
    i = pl.program_id(0)
    n = pl.num_programs(0)
    nbuf, bs, _ = buf.shape

    @pl.when(i == 0)
    def _prologue():
        for b in range(nbuf):
            @pl.when(b < n)
            def _start():
                pltpu.make_async_copy(
                    adj_hbm.at[pl.ds(b * bs, bs), :], buf.at[b], sems.at[b]
                ).start()

    slot = jax.lax.rem(i, nbuf)
    pltpu.make_async_copy(
        adj_hbm.at[pl.ds(i * bs, bs), :], buf.at[slot], sems.at[slot]
    ).wait()
    o_ref[...] = jnp.dot(
        buf[slot], x_ref[...], preferred_element_type=jnp.float32
    )
    nxt = i + nbuf

    @pl.when(nxt < n)
    def _prefetch():
        pltpu.make_async_copy(
            adj_hbm.at[pl.ds(nxt * bs, bs), :], buf.at[slot], sems.at[slot]
        ).start()


def kernel(x, adj):
    m, k = adj.shape
    _, d = x.shape
    bs = _pick_block(m, (200, 80, 40, 16, 8))
    nsteps = m // bs
    return pl.pallas_call(
        _mm_stream_kernel,
        grid=(nsteps,),
        in_specs=[
            pl.BlockSpec(memory_space=pltpu.MemorySpace.HBM),
            pl.BlockSpec((k, d), lambda i: (0, 0)),
        ],
        out_specs=pl.BlockSpec((bs, d), lambda i: (i, 0)),
        out_shape=jax.ShapeDtypeStruct((m, d), jnp.float32),
        scratch_shapes=[
            pltpu.VMEM((_NBUF, bs, k), jnp.float32),
            pltpu.SemaphoreType.DMA((_NBUF,)),
        ],
        compiler_params=pltpu.CompilerParams(
            dimension_semantics=("arbitrary",),
        ),
    )(adj, x)


# speedup vs baseline: 1.0108x; 1.0108x over previous
import jax
import jax.numpy as jnp
from jax.experimental import pallas as pl
from jax.experimental.pallas import tpu as pltpu


def _pick_block(dim, preferred):
    for b in preferred:
        if dim % b == 0:
            return b
    return dim


def _mm_kernel(adj_ref, x_ref, o_ref):
    o_ref[...] = jnp.dot(
        adj_ref[...], x_ref[...], preferred_element_type=jnp.float32
    )


def kernel(x, adj):
    m, k = adj.shape
    _, d = x.shape
    bm = _pick_block(m, (400, 200, 80, 40, 16, 8))
    return pl.pallas_call(
        _mm_kernel,
        grid=(m // bm,),
        in_specs=[
            pl.BlockSpec((bm, k), lambda i: (i, 0)),
            pl.BlockSpec((k, d), lambda i: (0, 0)),
        ],
        out_specs=pl.BlockSpec((bm, d), lambda i: (i, 0)),
        out_shape=jax.ShapeDtypeStruct((m, d), jnp.float32),
        compiler_params=pltpu.CompilerParams(
            dimension_semantics=("parallel",),
        ),
    )(adj, x)


# speedup vs baseline: 1.0168x; 1.0060x over previous
import jax
import jax.numpy as jnp
from jax.experimental import pallas as pl
from jax.experimental.pallas import tpu as pltpu


def _pick_block(dim, preferred):
    for b in preferred:
        if dim % b == 0:
            return b
    return dim


def _mm_kernel(adj_ref, x_ref, o_ref):
    o_ref[...] = jnp.dot(
        adj_ref[...], x_ref[...], preferred_element_type=jnp.float32
    )


def kernel(x, adj):
    m, k = adj.shape
    _, d = x.shape
    bm = _pick_block(m, (200, 80, 40, 16, 8))
    return pl.pallas_call(
        _mm_kernel,
        grid=(m // bm,),
        in_specs=[
            pl.BlockSpec((bm, k), lambda i: (i, 0)),
            pl.BlockSpec((k, d), lambda i: (0, 0)),
        ],
        out_specs=pl.BlockSpec((bm, d), lambda i: (i, 0)),
        out_shape=jax.ShapeDtypeStruct((m, d), jnp.float32),
        compiler_params=pltpu.CompilerParams(
            dimension_semantics=("parallel",),
        ),
    )(adj, x)
